# baseline (speedup 1.0000x reference)
import functools

import jax
import jax.numpy as jnp
from jax import lax
from jax.experimental import pallas as pl
from jax.experimental.pallas import tpu as pltpu
from jax.experimental.pallas import tpu_sc as plsc

EXPERTS = 16
HIDDEN = 2048
TOKENS = 8192
BLOCK = 2048

NCHUNK = 2
CTOK = TOKENS // NCHUNK

NC = 2
NS = 16
NW = NC * NS
TPW = CTOK // NW


def _logits_body(x_ref, w_ref, b_ref, logits_ref):
    logits_ref[:] = lax.dot_general(
        x_ref[:], w_ref[:], (((1,), (1,)), ((), ())),
        preferred_element_type=jnp.float32,
    ) + b_ref[:]


def _logits_tc(x, gate_w, gate_b2d):
    return pl.pallas_call(
        _logits_body,
        grid=(CTOK // BLOCK,),
        in_specs=[
            pl.BlockSpec((BLOCK, HIDDEN), lambda i: (i, 0)),
            pl.BlockSpec((EXPERTS, HIDDEN), lambda i: (0, 0)),
            pl.BlockSpec((1, EXPERTS), lambda i: (0, 0)),
        ],
        out_specs=pl.BlockSpec((BLOCK, EXPERTS), lambda i: (i, 0)),
        out_shape=jax.ShapeDtypeStruct((CTOK, EXPERTS), jnp.float32),
    )(x, gate_w, gate_b2d)


@functools.partial(
    pl.kernel,
    mesh=plsc.VectorSubcoreMesh(core_axis_name="c", subcore_axis_name="s"),
    out_type=[
        jax.ShapeDtypeStruct((CTOK * EXPERTS,), jnp.float32),
        jax.ShapeDtypeStruct((CTOK * 2,), jnp.int32),
    ],
    scratch_types=[
        pltpu.VMEM((TPW * EXPERTS,), jnp.float32),
        pltpu.VMEM((TPW * EXPERTS,), jnp.float32),
        pltpu.VMEM((TPW * 2,), jnp.int32),
    ],
)
def _route_sc(logits_hbm, sparse_hbm, idx_hbm, lg_v, sp_v, ix_v):
    wid = lax.axis_index("s") * NC + lax.axis_index("c")
    base = wid * TPW
    pltpu.sync_copy(logits_hbm.at[pl.ds(base * EXPERTS, TPW * EXPERTS)], lg_v)

    col = lax.broadcasted_iota(jnp.int32, (EXPERTS,), 0)
    neg_inf = jnp.float32(-jnp.inf)

    def argmax2(v):
        m, i = v, col
        for s in (8, 4, 2, 1):
            mg = m.at[col ^ s].get(mode="promise_in_bounds")
            ig = i.at[col ^ s].get(mode="promise_in_bounds")
            take = (mg > m) | ((mg == m) & (ig < i))
            m = jnp.where(take, mg, m)
            i = jnp.where(take, ig, i)
        return m, i

    def group(g, carry):
        ivec = jnp.zeros((EXPERTS,), jnp.int32)
        for k in range(8):
            t = g * 8 + k
            v = lg_v[pl.ds(t * EXPERTS, EXPERTS)]
            m1, i1 = argmax2(v)
            hit1 = col == i1
            m2, i2 = argmax2(jnp.where(hit1, neg_inf, v))
            hit2 = col == i2
            e2 = jnp.exp(m2 - m1)
            p1 = 1.0 / (1.0 + e2)
            p2 = e2 * p1
            sp_v[pl.ds(t * EXPERTS, EXPERTS)] = jnp.where(
                hit1, p1, jnp.where(hit2, p2, jnp.float32(0.0)))
            ivec = jnp.where(col == 2 * k, i1, ivec)
            ivec = jnp.where(col == 2 * k + 1, i2, ivec)
        ix_v[pl.ds(g * EXPERTS, EXPERTS)] = ivec
        return carry

    lax.fori_loop(0, TPW // 8, group, 0)
    pltpu.sync_copy(sp_v, sparse_hbm.at[pl.ds(base * EXPERTS, TPW * EXPERTS)])
    pltpu.sync_copy(ix_v, idx_hbm.at[pl.ds(base * 2, TPW * 2)])


def kernel(x, gate_w, gate_b):
    b2d = gate_b.reshape(1, EXPERTS)
    logits_c, sparse_c, idx_c = [], [], []
    for c in range(NCHUNK):
        lg = _logits_tc(x[c * CTOK:(c + 1) * CTOK], gate_w, b2d)
        sp, ix = _route_sc(lg.reshape(-1))
        logits_c.append(lg)
        sparse_c.append(sp.reshape(CTOK, EXPERTS))
        idx_c.append(ix.reshape(CTOK, 2))
    return (jnp.concatenate(sparse_c),
            jnp.concatenate(idx_c),
            jnp.concatenate(logits_c))


# speedup vs baseline: 2.8580x; 2.8580x over previous
import jax
import jax.numpy as jnp
from jax.experimental import pallas as pl

EXPERTS = 16
HIDDEN = 2048
TOKENS = 8192
BLOCK = 2048


def _gating_body(x_ref, w_ref, b_ref, sparse_ref, idx_ref, logits_ref):
    logits = jax.lax.dot_general(
        x_ref[:], w_ref[:], (((1,), (1,)), ((), ())),
        preferred_element_type=jnp.float32,
    ) + b_ref[:]
    logits_ref[:] = logits

    col = jax.lax.broadcasted_iota(jnp.int32, logits.shape, 1)
    m1 = jnp.max(logits, axis=1, keepdims=True)
    i1 = jnp.min(jnp.where(logits == m1, col, EXPERTS), axis=1, keepdims=True)
    masked = jnp.where(col == i1, -jnp.inf, logits)
    m2 = jnp.max(masked, axis=1, keepdims=True)
    i2 = jnp.min(jnp.where(masked == m2, col, EXPERTS), axis=1, keepdims=True)

    e2 = jnp.exp(m2 - m1)
    denom = 1.0 + e2
    sparse_ref[:] = jnp.where(col == i1, 1.0 / denom,
                              jnp.where(col == i2, e2 / denom, 0.0))
    idx_ref[:] = jnp.concatenate([i1, i2], axis=1)


def kernel(x, gate_w, gate_b):
    grid = (TOKENS // BLOCK,)
    sparse, idx, logits = pl.pallas_call(
        _gating_body,
        grid=grid,
        in_specs=[
            pl.BlockSpec((BLOCK, HIDDEN), lambda i: (i, 0)),
            pl.BlockSpec((EXPERTS, HIDDEN), lambda i: (0, 0)),
            pl.BlockSpec((1, EXPERTS), lambda i: (0, 0)),
        ],
        out_specs=[
            pl.BlockSpec((BLOCK, EXPERTS), lambda i: (i, 0)),
            pl.BlockSpec((BLOCK, 2), lambda i: (i, 0)),
            pl.BlockSpec((BLOCK, EXPERTS), lambda i: (i, 0)),
        ],
        out_shape=[
            jax.ShapeDtypeStruct((TOKENS, EXPERTS), jnp.float32),
            jax.ShapeDtypeStruct((TOKENS, 2), jnp.int32),
            jax.ShapeDtypeStruct((TOKENS, EXPERTS), jnp.float32),
        ],
    )(x, gate_w, gate_b.reshape(1, EXPERTS))
    return (sparse, idx, logits)
